# Initial kernel scaffold; baseline (speedup 1.0000x reference)
import jax
import jax.numpy as jnp
from jax.experimental import pallas as pl


def kernel(logits):
    raise NotImplementedError("write your pallas kernel here")


# speedup vs baseline: 23.0410x; 23.0410x over previous
import jax
import jax.numpy as jnp
from jax.experimental import pallas as pl

_ROWS_PER_BLOCK = 8
_NEWTON_ITERS = 16


def _entmax15_block(x_ref, o_ref):
    x = x_ref[...]
    m = jnp.max(x, axis=-1, keepdims=True)
    xn = (x - m) * 0.5

    def body(_, tau):
        p = jnp.maximum(xn - tau, 0.0)
        f = jnp.sum(p * p, axis=-1, keepdims=True)
        s = jnp.sum(p, axis=-1, keepdims=True)
        return tau + (f - 1.0) / jnp.maximum(2.0 * s, 1e-30)

    tau0 = jnp.full((x.shape[0], 1), -1.0, dtype=x.dtype)
    tau = jax.lax.fori_loop(0, _NEWTON_ITERS, body, tau0)
    t = jnp.maximum(xn - tau, 0.0)
    o_ref[...] = t * t


def kernel(logits):
    rows, cols = logits.shape
    grid = (rows // _ROWS_PER_BLOCK,)
    return pl.pallas_call(
        _entmax15_block,
        grid=grid,
        in_specs=[
            pl.BlockSpec((_ROWS_PER_BLOCK, cols), lambda i: (i, 0)),
        ],
        out_specs=pl.BlockSpec((_ROWS_PER_BLOCK, cols), lambda i: (i, 0)),
        out_shape=jax.ShapeDtypeStruct((rows, cols), logits.dtype),
    )(logits)


# speedup vs baseline: 29.5785x; 1.2837x over previous
import jax
import jax.numpy as jnp
from jax.experimental import pallas as pl

_ROWS_PER_BLOCK = 8
_NEWTON_ITERS = 12


def _entmax15_block(x_ref, o_ref):
    x = x_ref[...]
    m = jnp.max(x, axis=-1, keepdims=True)
    xn = (x - m) * 0.5

    def body(_, tau):
        p = jnp.maximum(xn - tau, 0.0)
        f = jnp.sum(p * p, axis=-1, keepdims=True)
        s = jnp.sum(p, axis=-1, keepdims=True)
        return tau + (f - 1.0) / jnp.maximum(2.0 * s, 1e-30)

    tau0 = jnp.full((x.shape[0], 1), -1.0, dtype=x.dtype)
    tau = jax.lax.fori_loop(0, _NEWTON_ITERS, body, tau0)
    t = jnp.maximum(xn - tau, 0.0)
    o_ref[...] = t * t


def kernel(logits):
    rows, cols = logits.shape
    grid = (rows // _ROWS_PER_BLOCK,)
    return pl.pallas_call(
        _entmax15_block,
        grid=grid,
        in_specs=[
            pl.BlockSpec((_ROWS_PER_BLOCK, cols), lambda i: (i, 0)),
        ],
        out_specs=pl.BlockSpec((_ROWS_PER_BLOCK, cols), lambda i: (i, 0)),
        out_shape=jax.ShapeDtypeStruct((rows, cols), logits.dtype),
    )(logits)
